# baseline (speedup 1.0000x reference)
import functools
import math

import jax
import jax.numpy as jnp
from jax import lax
from jax.experimental import pallas as pl
from jax.experimental.pallas import tpu as pltpu
from jax.experimental.pallas import tpu_sc as plsc

N = 10000
E = 320000
D = 128
H = 4
C = 32
DE = 16
HC = H * C
_INV_SQRT_C = 1.0 / math.sqrt(C)

_QQW = 196
_KW = 132
_EAW = 20
ACCW = 100

_ROWS_BLK = 2000
_EA_BLK = 10000

_PERM = [C * (j % H) + j // H for j in range(HC)]


def _pre_body(x_ref, wq_ref, bq_ref, wk_ref, bk_ref, r_ref, qq_ref, k_ref):
    x = x_ref[...]
    q = jnp.dot(x, wq_ref[...], preferred_element_type=jnp.float32) + bq_ref[...]
    k = jnp.dot(x, wk_ref[...], preferred_element_type=jnp.float32) + bk_ref[...]
    qwe = jnp.dot(q, r_ref[...], preferred_element_type=jnp.float32)
    nb = q.shape[0]
    qq_ref[...] = jnp.concatenate(
        [q, qwe, jnp.zeros((nb, _QQW - HC - H * DE), jnp.float32)], axis=1)
    k_ref[...] = k


def _pre_tables(x, Wq, bq, Wk, bk, We):
    perm = jnp.asarray(_PERM, jnp.int32)
    wq_p = Wq[:, perm]
    bq_p = bq[perm].reshape(1, HC)
    wk_p = jnp.concatenate(
        [Wk[:, perm], jnp.zeros((D, _KW - HC), jnp.float32)], axis=1)
    bk_p = jnp.concatenate(
        [bk[perm], jnp.zeros((_KW - HC,), jnp.float32)]).reshape(1, _KW)
    we_r = We.reshape(DE, H, C)
    rt = jnp.zeros((C, H, DE, H), jnp.float32)
    for h in range(H):
        rt = rt.at[:, h, :, h].set(we_r[:, h, :].T)
    r = rt.reshape(HC, H * DE)
    grid = (N // _ROWS_BLK,)
    return pl.pallas_call(
        _pre_body,
        grid=grid,
        in_specs=[
            pl.BlockSpec((_ROWS_BLK, D), lambda i: (i, 0)),
            pl.BlockSpec((D, HC), lambda i: (0, 0)),
            pl.BlockSpec((1, HC), lambda i: (0, 0)),
            pl.BlockSpec((D, _KW), lambda i: (0, 0)),
            pl.BlockSpec((1, _KW), lambda i: (0, 0)),
            pl.BlockSpec((HC, H * DE), lambda i: (0, 0)),
        ],
        out_specs=[
            pl.BlockSpec((_ROWS_BLK, _QQW), lambda i: (i, 0)),
            pl.BlockSpec((_ROWS_BLK, _KW), lambda i: (i, 0)),
        ],
        out_shape=[
            jax.ShapeDtypeStruct((N, _QQW), jnp.float32),
            jax.ShapeDtypeStruct((N, _KW), jnp.float32),
        ],
    )(x, wq_p, bq_p, wk_p, bk_p, r)


def _ea_pad_body(ea_ref, out_ref):
    ea = ea_ref[...]
    out_ref[...] = jnp.concatenate([ea, ea[:, 0:_EAW - DE]], axis=1)


def _ea_pad(edge_attr):
    grid = (E // _EA_BLK,)
    return pl.pallas_call(
        _ea_pad_body,
        grid=grid,
        in_specs=[pl.BlockSpec((_EA_BLK, DE), lambda i: (i, 0))],
        out_specs=pl.BlockSpec((_EA_BLK, _EAW), lambda i: (i, 0)),
        out_shape=jax.ShapeDtypeStruct((E, _EAW), jnp.float32),
    )(edge_attr)


def _post_body(acc_ref, x_ref, we_ref, wv_ref, bv_ref, wskip_ref, bskip_ref,
               g1_ref, g2_ref, bg_ref, out_ref):
    a = acc_ref[0] + acc_ref[1]
    x = x_ref[...]
    v = jnp.dot(x, wv_ref[...], preferred_element_type=jnp.float32) + bv_ref[...]
    skip = jnp.dot(x, wskip_ref[...], preferred_element_type=jnp.float32) + bskip_ref[...]
    parts = []
    for h in range(H):
        b0 = 16 + 21 * h
        if h == 0:
            s_h = a[:, b0:b0 + DE]
        else:
            s_h = jnp.concatenate(
                [a[:, b0 + DE:b0 + DE + h], a[:, b0 + h:b0 + DE]], axis=1)
        we_h = we_ref[:, C * h:C * h + C]
        accum_h = jnp.dot(s_h, we_h, preferred_element_type=jnp.float32)
        d_h = a[:, h:h + 1]
        parts.append((v[:, C * h:C * h + C] * d_h + accum_h) / (d_h + 1e-16))
    nf = jnp.concatenate(parts, axis=1)
    glin = (jnp.sum(nf * g1_ref[...], axis=1, keepdims=True)
            + jnp.sum(skip * g2_ref[...], axis=1, keepdims=True)
            + bg_ref[...])
    g = jax.nn.sigmoid(glin)
    out_ref[...] = g * skip + (1.0 - g) * nf


def _post(acc, x, We, Wv, bv, Wskip, bskip, Wgate, bgate):
    g1 = (Wgate[0:HC, 0] - Wgate[2 * HC:3 * HC, 0]).reshape(1, HC)
    g2 = (Wgate[HC:2 * HC, 0] + Wgate[2 * HC:3 * HC, 0]).reshape(1, HC)
    grid = (N // _ROWS_BLK,)
    return pl.pallas_call(
        _post_body,
        grid=grid,
        in_specs=[
            pl.BlockSpec((2, _ROWS_BLK, ACCW), lambda i: (0, i, 0)),
            pl.BlockSpec((_ROWS_BLK, D), lambda i: (i, 0)),
            pl.BlockSpec((DE, HC), lambda i: (0, 0)),
            pl.BlockSpec((D, HC), lambda i: (0, 0)),
            pl.BlockSpec((1, HC), lambda i: (0, 0)),
            pl.BlockSpec((D, HC), lambda i: (0, 0)),
            pl.BlockSpec((1, HC), lambda i: (0, 0)),
            pl.BlockSpec((1, HC), lambda i: (0, 0)),
            pl.BlockSpec((1, HC), lambda i: (0, 0)),
            pl.BlockSpec((1, 1), lambda i: (0, 0)),
        ],
        out_specs=pl.BlockSpec((_ROWS_BLK, HC), lambda i: (i, 0)),
        out_shape=jax.ShapeDtypeStruct((N, HC), jnp.float32),
    )(acc, x, We, Wv, bv.reshape(1, HC), Wskip, bskip.reshape(1, HC),
      g1, g2, bgate.reshape(1, 1))


_B = 80
_NW = 32
_EPW = E // _NW
_NCH = _EPW // _B
N_ACC = 10240
_RPT = N_ACC // 16


def _sc_edge_body(qq_hbm, k_hbm, src_hbm, trg_hbm, ea_hbm, zero_hbm, out_hbm,
                  src_v, trg_v, qq_v, k_v, ea_v, pay_v, acc_sh,
                  sem0, sem1, sem2):
    c = lax.axis_index("c")
    s = lax.axis_index("s")
    row0 = s * _RPT
    pltpu.sync_copy(zero_hbm.at[pl.ds(row0, _RPT)],
                    acc_sh.at[pl.ds(row0, _RPT)])
    plsc.subcore_barrier()
    lane = lax.iota(jnp.int32, 16)
    eo = lane >> 2
    hh = lane & 3
    col21 = 21 * hh

    def zrow(i, carry):
        z = jnp.zeros((16,), jnp.float32)
        for off in (0, 16, 32, 48, 64, 80, 84):
            pay_v[i, pl.ds(off, 16)] = z
        return carry

    lax.fori_loop(0, _B, zrow, 0)
    base0 = c * (E // 2) + s * _EPW

    def chunk_body(i, carry):
        base = base0 + i * _B
        pltpu.sync_copy(src_hbm.at[pl.ds(base, _B)], src_v)
        pltpu.sync_copy(trg_hbm.at[pl.ds(base, _B)], trg_v)
        cp0 = pltpu.async_copy(qq_hbm.at[src_v], qq_v, sem0)
        cp1 = pltpu.async_copy(k_hbm.at[trg_v], k_v, sem1)
        cp2 = pltpu.async_copy(ea_hbm.at[pl.ds(base, _B)], ea_v, sem2)
        cp0.wait()
        cp1.wait()
        cp2.wait()

        zv = jnp.zeros((16,), jnp.int32)

        @plsc.parallel_loop(0, _B // 4, unroll=1)
        def group_body(g):
            row = g * 4 + eo
            fq = row * _QQW + hh
            fk = row * _KW + hh
            fe = row * _EAW + hh
            acc = jnp.zeros((16,), jnp.float32)
            for j in range(C):
                qv = plsc.load_gather(qq_v, [zv, fq])
                kv = plsc.load_gather(k_v, [zv, fk])
                acc = acc + qv * kv
                fq = fq + 4
                fk = fk + 4
            fw = row * _QQW + (HC + hh)
            for d in range(DE):
                qwe = plsc.load_gather(qq_v, [zv, fw])
                eav = plsc.load_gather(ea_v, [zv, fe])
                acc = acc + qwe * eav
                fw = fw + 4
                fe = fe + 1
            w = jnp.exp(acc * _INV_SQRT_C)
            plsc.store_scatter(pay_v, [row, hh], w)
            fe2 = row * _EAW + hh
            fs = row * ACCW + (col21 + 16)
            for d in range(DE):
                eav = plsc.load_gather(ea_v, [zv, fe2])
                plsc.store_scatter(pay_v, [zv, fs], w * eav)
                fe2 = fe2 + 1
                fs = fs + 1

        pltpu.sync_copy(pay_v, acc_sh.at[trg_v], add=True)
        return carry

    lax.fori_loop(0, _NCH, chunk_body, 0)
    plsc.subcore_barrier()
    pltpu.sync_copy(acc_sh.at[pl.ds(row0, _RPT)],
                    out_hbm.at[c, pl.ds(row0, _RPT)])


_sc_edge = functools.partial(
    pl.kernel,
    mesh=plsc.VectorSubcoreMesh(core_axis_name="c", subcore_axis_name="s"),
    out_type=jax.ShapeDtypeStruct((2, N_ACC, ACCW), jnp.float32),
    compiler_params=pltpu.CompilerParams(
        needs_layout_passes=False, use_tc_tiling_on_sc=False),
    scratch_types=[
        pltpu.VMEM((_B,), jnp.int32),
        pltpu.VMEM((_B,), jnp.int32),
        pltpu.VMEM((_B, _QQW), jnp.float32),
        pltpu.VMEM((_B, _KW), jnp.float32),
        pltpu.VMEM((_B, _EAW), jnp.float32),
        pltpu.VMEM((_B, ACCW), jnp.float32),
        pltpu.VMEM_SHARED((N_ACC, ACCW), jnp.float32),
        pltpu.SemaphoreType.DMA,
        pltpu.SemaphoreType.DMA,
        pltpu.SemaphoreType.DMA,
    ],
)(_sc_edge_body)


def _edge_pass(qq, ktab, edge_indices, eap):
    src = edge_indices[0]
    trg = edge_indices[1]
    zeros = jnp.zeros((N_ACC, ACCW), jnp.float32)
    return _sc_edge(qq, ktab, src, trg, eap, zeros)


def kernel(x, edge_indices, edge_attr, Wq, bq, Wk, bk, We, Wv, bv,
           Wskip, bskip, Wgate, bgate):
    qq, ktab = _pre_tables(x, Wq, bq, Wk, bk, We)
    eap = _ea_pad(edge_attr)
    acc = _edge_pass(qq, ktab, edge_indices, eap)
    return _post(acc, x, We, Wv, bv, Wskip, bskip, Wgate, bgate)


# speedup vs baseline: 1.2379x; 1.2379x over previous
import functools
import math

import jax
import jax.numpy as jnp
from jax import lax
from jax.experimental import pallas as pl
from jax.experimental.pallas import tpu as pltpu
from jax.experimental.pallas import tpu_sc as plsc

N = 10000
E = 320000
D = 128
H = 4
C = 32
DE = 16
HC = H * C
ACCW = 80
_INV_SQRT_C = 1.0 / math.sqrt(C)

_ROWS_BLK = 2000


def _pre_body(x_ref, wq_ref, bq_ref, wk_ref, bk_ref, we_ref, qq_ref, k_ref):
    x = x_ref[...]
    q = jnp.dot(x, wq_ref[...], preferred_element_type=jnp.float32) + bq_ref[...]
    k = jnp.dot(x, wk_ref[...], preferred_element_type=jnp.float32) + bk_ref[...]
    qq_ref[:, 0:HC] = q
    for h in range(H):
        qh = q[:, C * h:C * h + C]
        we_h = we_ref[:, C * h:C * h + C]
        qq_ref[:, HC + DE * h:HC + DE * (h + 1)] = jax.lax.dot_general(
            qh, we_h, (((1,), (1,)), ((), ())),
            preferred_element_type=jnp.float32)
    k_ref[...] = k


def _pre_tables(x, Wq, bq, Wk, bk, We):
    grid = (N // _ROWS_BLK,)
    return pl.pallas_call(
        _pre_body,
        grid=grid,
        in_specs=[
            pl.BlockSpec((_ROWS_BLK, D), lambda i: (i, 0)),
            pl.BlockSpec((D, HC), lambda i: (0, 0)),
            pl.BlockSpec((1, HC), lambda i: (0, 0)),
            pl.BlockSpec((D, HC), lambda i: (0, 0)),
            pl.BlockSpec((1, HC), lambda i: (0, 0)),
            pl.BlockSpec((DE, HC), lambda i: (0, 0)),
        ],
        out_specs=[
            pl.BlockSpec((_ROWS_BLK, HC + H * DE), lambda i: (i, 0)),
            pl.BlockSpec((_ROWS_BLK, HC), lambda i: (i, 0)),
        ],
        out_shape=[
            jax.ShapeDtypeStruct((N, HC + H * DE), jnp.float32),
            jax.ShapeDtypeStruct((N, HC), jnp.float32),
        ],
    )(x, Wq, bq.reshape(1, HC), Wk, bk.reshape(1, HC), We)


def _post_body(acc_ref, x_ref, we_ref, wv_ref, bv_ref, wskip_ref, bskip_ref,
               g1_ref, g2_ref, bg_ref, out_ref):
    a = acc_ref[0] + acc_ref[1]
    x = x_ref[...]
    v = jnp.dot(x, wv_ref[...], preferred_element_type=jnp.float32) + bv_ref[...]
    skip = jnp.dot(x, wskip_ref[...], preferred_element_type=jnp.float32) + bskip_ref[...]
    parts = []
    for h in range(H):
        s_h = a[:, 16 + DE * h:16 + DE * (h + 1)]
        we_h = we_ref[:, C * h:C * h + C]
        accum_h = jnp.dot(s_h, we_h, preferred_element_type=jnp.float32)
        d_h = a[:, h:h + 1]
        parts.append((v[:, C * h:C * h + C] * d_h + accum_h) / (d_h + 1e-16))
    nf = jnp.concatenate(parts, axis=1)
    glin = (jnp.sum(nf * g1_ref[...], axis=1, keepdims=True)
            + jnp.sum(skip * g2_ref[...], axis=1, keepdims=True)
            + bg_ref[...])
    g = jax.nn.sigmoid(glin)
    out_ref[...] = g * skip + (1.0 - g) * nf


def _post(acc, x, We, Wv, bv, Wskip, bskip, Wgate, bgate):
    g1 = (Wgate[0:HC, 0] - Wgate[2 * HC:3 * HC, 0]).reshape(1, HC)
    g2 = (Wgate[HC:2 * HC, 0] + Wgate[2 * HC:3 * HC, 0]).reshape(1, HC)
    grid = (N // _ROWS_BLK,)
    return pl.pallas_call(
        _post_body,
        grid=grid,
        in_specs=[
            pl.BlockSpec((2, _ROWS_BLK, ACCW), lambda i: (0, i, 0)),
            pl.BlockSpec((_ROWS_BLK, D), lambda i: (i, 0)),
            pl.BlockSpec((DE, HC), lambda i: (0, 0)),
            pl.BlockSpec((D, HC), lambda i: (0, 0)),
            pl.BlockSpec((1, HC), lambda i: (0, 0)),
            pl.BlockSpec((D, HC), lambda i: (0, 0)),
            pl.BlockSpec((1, HC), lambda i: (0, 0)),
            pl.BlockSpec((1, HC), lambda i: (0, 0)),
            pl.BlockSpec((1, HC), lambda i: (0, 0)),
            pl.BlockSpec((1, 1), lambda i: (0, 0)),
        ],
        out_specs=pl.BlockSpec((_ROWS_BLK, HC), lambda i: (i, 0)),
        out_shape=jax.ShapeDtypeStruct((N, HC), jnp.float32),
    )(acc, x, We, Wv, bv.reshape(1, HC), Wskip, bskip.reshape(1, HC),
      g1, g2, bgate.reshape(1, 1))


_B = 80
_NW = 32
_EPW = E // _NW
_NCH = _EPW // _B
N_ACC = 10240
_RPT = N_ACC // 16
_QQW = HC + H * DE


def _sc_edge_body(qq_hbm, k_hbm, src_hbm, trg_hbm, ea_hbm, zero_hbm, out_hbm,
                  src_v, trg_v, qq_v, k_v, ea_v, pay_v, acc_sh,
                  sem0, sem1, sem2):
    c = lax.axis_index("c")
    s = lax.axis_index("s")
    row0 = s * _RPT
    pltpu.sync_copy(zero_hbm.at[pl.ds(row0, _RPT)],
                    acc_sh.at[pl.ds(row0, _RPT)])
    plsc.subcore_barrier()
    lane = lax.iota(jnp.int32, 16)
    eo = lane >> 2
    hh = lane & 3
    colqk0 = hh * C
    colwe0 = HC + hh * DE
    paycol0 = 16 + hh * DE

    def zrow(i, carry):
        z = jnp.zeros((16,), jnp.float32)
        for off in (0, 16, 32, 48, 64):
            pay_v[i, pl.ds(off, 16)] = z
        return carry

    lax.fori_loop(0, _B, zrow, 0)
    base0 = c * (E // 2) + s * _EPW

    def chunk_body(i, carry):
        base = base0 + i * _B
        pltpu.sync_copy(src_hbm.at[pl.ds(base, _B)], src_v)
        pltpu.sync_copy(trg_hbm.at[pl.ds(base, _B)], trg_v)
        cp0 = pltpu.async_copy(qq_hbm.at[src_v], qq_v, sem0)
        cp1 = pltpu.async_copy(k_hbm.at[trg_v], k_v, sem1)
        cp2 = pltpu.async_copy(ea_hbm.at[pl.ds(base, _B)], ea_v, sem2)
        cp0.wait()
        cp1.wait()
        cp2.wait()

        @plsc.parallel_loop(0, _B // 4, unroll=1)
        def group_body(g):
            row = g * 4 + eo
            acc = jnp.zeros((16,), jnp.float32)
            s32 = lane + 0
            for j in range(C):
                cj = colqk0 + s32
                qv = plsc.load_gather(qq_v, [row, cj])
                kv = plsc.load_gather(k_v, [row, cj])
                acc = acc + qv * kv
                s32 = (s32 + 1) & (C - 1)
            s16 = lane & (DE - 1)
            for d in range(DE):
                qwe = plsc.load_gather(qq_v, [row, colwe0 + s16])
                eav = plsc.load_gather(ea_v, [row, s16])
                acc = acc + qwe * eav
                s16 = (s16 + 1) & (DE - 1)
            w = jnp.exp(acc * _INV_SQRT_C)
            plsc.store_scatter(pay_v, [row, hh], w)
            s16 = lane & (DE - 1)
            for d in range(DE):
                eav = plsc.load_gather(ea_v, [row, s16])
                plsc.store_scatter(pay_v, [row, paycol0 + s16], w * eav)
                s16 = (s16 + 1) & (DE - 1)

        pltpu.sync_copy(pay_v, acc_sh.at[trg_v], add=True)
        return carry

    lax.fori_loop(0, _NCH, chunk_body, 0)
    plsc.subcore_barrier()
    pltpu.sync_copy(acc_sh.at[pl.ds(row0, _RPT)],
                    out_hbm.at[c, pl.ds(row0, _RPT)])


_sc_edge = functools.partial(
    pl.kernel,
    mesh=plsc.VectorSubcoreMesh(core_axis_name="c", subcore_axis_name="s"),
    out_type=jax.ShapeDtypeStruct((2, N_ACC, ACCW), jnp.float32),
    compiler_params=pltpu.CompilerParams(
        needs_layout_passes=False, use_tc_tiling_on_sc=False),
    scratch_types=[
        pltpu.VMEM((_B,), jnp.int32),
        pltpu.VMEM((_B,), jnp.int32),
        pltpu.VMEM((_B, _QQW), jnp.float32),
        pltpu.VMEM((_B, HC), jnp.float32),
        pltpu.VMEM((_B, DE), jnp.float32),
        pltpu.VMEM((_B, ACCW), jnp.float32),
        pltpu.VMEM_SHARED((N_ACC, ACCW), jnp.float32),
        pltpu.SemaphoreType.DMA,
        pltpu.SemaphoreType.DMA,
        pltpu.SemaphoreType.DMA,
    ],
)(_sc_edge_body)


def _edge_pass(qq, ktab, edge_indices, edge_attr):
    src = edge_indices[0]
    trg = edge_indices[1]
    zeros = jnp.zeros((N_ACC, ACCW), jnp.float32)
    return _sc_edge(qq, ktab, src, trg, edge_attr, zeros)


def kernel(x, edge_indices, edge_attr, Wq, bq, Wk, bk, We, Wv, bv,
           Wskip, bskip, Wgate, bgate):
    qq, ktab = _pre_tables(x, Wq, bq, Wk, bk, We)
    acc = _edge_pass(qq, ktab, edge_indices, edge_attr)
    return _post(acc, x, We, Wv, bv, Wskip, bskip, Wgate, bgate)


# speedup vs baseline: 1.5648x; 1.2641x over previous
import functools
import math

import jax
import jax.numpy as jnp
from jax import lax
from jax.experimental import pallas as pl
from jax.experimental.pallas import tpu as pltpu
from jax.experimental.pallas import tpu_sc as plsc

N = 10000
E = 320000
D = 128
H = 4
C = 32
DE = 16
HC = H * C
ACCW = 80
_INV_SQRT_C = 1.0 / math.sqrt(C)

_ROWS_BLK = 2000


def _pre_body(x_ref, wq_ref, bq_ref, wk_ref, bk_ref, we_ref, qq_ref, k_ref):
    x = x_ref[...]
    q = jnp.dot(x, wq_ref[...], preferred_element_type=jnp.float32) + bq_ref[...]
    k = jnp.dot(x, wk_ref[...], preferred_element_type=jnp.float32) + bk_ref[...]
    qq_ref[:, 0:HC] = q
    for h in range(H):
        qh = q[:, C * h:C * h + C]
        we_h = we_ref[:, C * h:C * h + C]
        qq_ref[:, HC + DE * h:HC + DE * (h + 1)] = jax.lax.dot_general(
            qh, we_h, (((1,), (1,)), ((), ())),
            preferred_element_type=jnp.float32)
    k_ref[...] = k


def _pre_tables(x, Wq, bq, Wk, bk, We):
    grid = (N // _ROWS_BLK,)
    return pl.pallas_call(
        _pre_body,
        grid=grid,
        in_specs=[
            pl.BlockSpec((_ROWS_BLK, D), lambda i: (i, 0)),
            pl.BlockSpec((D, HC), lambda i: (0, 0)),
            pl.BlockSpec((1, HC), lambda i: (0, 0)),
            pl.BlockSpec((D, HC), lambda i: (0, 0)),
            pl.BlockSpec((1, HC), lambda i: (0, 0)),
            pl.BlockSpec((DE, HC), lambda i: (0, 0)),
        ],
        out_specs=[
            pl.BlockSpec((_ROWS_BLK, HC + H * DE), lambda i: (i, 0)),
            pl.BlockSpec((_ROWS_BLK, HC), lambda i: (i, 0)),
        ],
        out_shape=[
            jax.ShapeDtypeStruct((N, HC + H * DE), jnp.float32),
            jax.ShapeDtypeStruct((N, HC), jnp.float32),
        ],
    )(x, Wq, bq.reshape(1, HC), Wk, bk.reshape(1, HC), We)


def _post_body(acc_ref, x_ref, we_ref, wv_ref, bv_ref, wskip_ref, bskip_ref,
               g1_ref, g2_ref, bg_ref, out_ref):
    a = acc_ref[0] + acc_ref[1]
    x = x_ref[...]
    v = jnp.dot(x, wv_ref[...], preferred_element_type=jnp.float32) + bv_ref[...]
    skip = jnp.dot(x, wskip_ref[...], preferred_element_type=jnp.float32) + bskip_ref[...]
    parts = []
    for h in range(H):
        s_h = a[:, 16 + DE * h:16 + DE * (h + 1)]
        we_h = we_ref[:, C * h:C * h + C]
        accum_h = jnp.dot(s_h, we_h, preferred_element_type=jnp.float32)
        d_h = a[:, h:h + 1]
        parts.append((v[:, C * h:C * h + C] * d_h + accum_h) / (d_h + 1e-16))
    nf = jnp.concatenate(parts, axis=1)
    glin = (jnp.sum(nf * g1_ref[...], axis=1, keepdims=True)
            + jnp.sum(skip * g2_ref[...], axis=1, keepdims=True)
            + bg_ref[...])
    g = jax.nn.sigmoid(glin)
    out_ref[...] = g * skip + (1.0 - g) * nf


def _post(acc, x, We, Wv, bv, Wskip, bskip, Wgate, bgate):
    g1 = (Wgate[0:HC, 0] - Wgate[2 * HC:3 * HC, 0]).reshape(1, HC)
    g2 = (Wgate[HC:2 * HC, 0] + Wgate[2 * HC:3 * HC, 0]).reshape(1, HC)
    grid = (N // _ROWS_BLK,)
    return pl.pallas_call(
        _post_body,
        grid=grid,
        in_specs=[
            pl.BlockSpec((2, _ROWS_BLK, ACCW), lambda i: (0, i, 0)),
            pl.BlockSpec((_ROWS_BLK, D), lambda i: (i, 0)),
            pl.BlockSpec((DE, HC), lambda i: (0, 0)),
            pl.BlockSpec((D, HC), lambda i: (0, 0)),
            pl.BlockSpec((1, HC), lambda i: (0, 0)),
            pl.BlockSpec((D, HC), lambda i: (0, 0)),
            pl.BlockSpec((1, HC), lambda i: (0, 0)),
            pl.BlockSpec((1, HC), lambda i: (0, 0)),
            pl.BlockSpec((1, HC), lambda i: (0, 0)),
            pl.BlockSpec((1, 1), lambda i: (0, 0)),
        ],
        out_specs=pl.BlockSpec((_ROWS_BLK, HC), lambda i: (i, 0)),
        out_shape=jax.ShapeDtypeStruct((N, HC), jnp.float32),
    )(acc, x, We, Wv, bv.reshape(1, HC), Wskip, bskip.reshape(1, HC),
      g1, g2, bgate.reshape(1, 1))


_B = 80
_NW = 32
_EPW = E // _NW
_NCH = _EPW // _B
N_ACC = 10240
_RPT = N_ACC // 16
_QQW = HC + H * DE


def _sc_edge_body(qq_hbm, k_hbm, src_hbm, trg_hbm, ea_hbm, zero_hbm, out_hbm,
                  src_v0, trg_v0, qq_v0, k_v0, ea_v0, pay_v0,
                  src_v1, trg_v1, qq_v1, k_v1, ea_v1, pay_v1, acc_sh,
                  gsem0, gsem1, ssem0, ssem1):
    c = lax.axis_index("c")
    s = lax.axis_index("s")
    row0 = s * _RPT
    pltpu.sync_copy(zero_hbm.at[pl.ds(row0, _RPT)],
                    acc_sh.at[pl.ds(row0, _RPT)])
    plsc.subcore_barrier()
    lane = lax.iota(jnp.int32, 16)
    eo = lane >> 2
    hh = lane & 3
    colqk0 = hh * C
    colwe0 = HC + hh * DE
    paycol0 = 16 + hh * DE

    def zrow(i, carry):
        z = jnp.zeros((16,), jnp.float32)
        for off in (0, 16, 32, 48, 64):
            pay_v0[i, pl.ds(off, 16)] = z
            pay_v1[i, pl.ds(off, 16)] = z
        return carry

    lax.fori_loop(0, _B, zrow, 0)
    base0 = c * (E // 2) + s * _EPW
    sets = ((src_v0, trg_v0, qq_v0, k_v0, ea_v0, pay_v0, gsem0, ssem0),
            (src_v1, trg_v1, qq_v1, k_v1, ea_v1, pay_v1, gsem1, ssem1))

    def issue_gathers(i, st):
        src_v, trg_v, qq_v, k_v, ea_v, _, gsem, _ = st
        base = base0 + i * _B
        pltpu.sync_copy(src_hbm.at[pl.ds(base, _B)], src_v)
        pltpu.sync_copy(trg_hbm.at[pl.ds(base, _B)], trg_v)
        pltpu.async_copy(qq_hbm.at[src_v], qq_v, gsem)
        pltpu.async_copy(k_hbm.at[trg_v], k_v, gsem)
        pltpu.async_copy(ea_hbm.at[pl.ds(base, _B)], ea_v, gsem)

    def drain_gathers(st):
        src_v, trg_v, qq_v, k_v, ea_v, _, gsem, _ = st
        pltpu.make_async_copy(qq_hbm.at[src_v], qq_v, gsem).wait()
        pltpu.make_async_copy(k_hbm.at[trg_v], k_v, gsem).wait()
        pltpu.make_async_copy(ea_hbm.at[pl.ds(0, _B)], ea_v, gsem).wait()

    def issue_scatter(st):
        _, trg_v, _, _, _, pay_v, _, ssem = st
        pltpu.async_copy(pay_v, acc_sh.at[trg_v], ssem, add=True)

    def drain_scatter(st):
        _, trg_v, _, _, _, pay_v, _, ssem = st
        pltpu.make_async_copy(pay_v, acc_sh.at[trg_v], ssem).wait()

    def compute(st):
        _, _, qq_v, k_v, ea_v, pay_v, _, _ = st

        @plsc.parallel_loop(0, _B // 4, unroll=1)
        def group_body(g):
            row = g * 4 + eo
            acc = jnp.zeros((16,), jnp.float32)
            s32 = lane + 0
            for j in range(C):
                cj = colqk0 + s32
                qv = plsc.load_gather(qq_v, [row, cj])
                kv = plsc.load_gather(k_v, [row, cj])
                acc = acc + qv * kv
                s32 = (s32 + 1) & (C - 1)
            s16 = lane & (DE - 1)
            for d in range(DE):
                qwe = plsc.load_gather(qq_v, [row, colwe0 + s16])
                eav = plsc.load_gather(ea_v, [row, s16])
                acc = acc + qwe * eav
                s16 = (s16 + 1) & (DE - 1)
            w = jnp.exp(acc * _INV_SQRT_C)
            plsc.store_scatter(pay_v, [row, hh], w)
            s16 = lane & (DE - 1)
            for d in range(DE):
                eav = plsc.load_gather(ea_v, [row, s16])
                plsc.store_scatter(pay_v, [row, paycol0 + s16], w * eav)
                s16 = (s16 + 1) & (DE - 1)

    def pair_body(t, carry):
        a = 2 * t

        @pl.when(t > 0)
        def _():
            drain_scatter(sets[0])
            drain_scatter(sets[1])

        issue_gathers(a, sets[0])
        issue_gathers(a + 1, sets[1])
        drain_gathers(sets[0])
        compute(sets[0])
        issue_scatter(sets[0])
        drain_gathers(sets[1])
        compute(sets[1])
        issue_scatter(sets[1])
        return carry

    lax.fori_loop(0, (_NCH - 1) // 2, pair_body, 0)
    drain_scatter(sets[0])
    issue_gathers(_NCH - 1, sets[0])
    drain_gathers(sets[0])
    drain_scatter(sets[1])
    compute(sets[0])
    issue_scatter(sets[0])
    drain_scatter(sets[0])
    plsc.subcore_barrier()
    pltpu.sync_copy(acc_sh.at[pl.ds(row0, _RPT)],
                    out_hbm.at[c, pl.ds(row0, _RPT)])


_sc_edge = functools.partial(
    pl.kernel,
    mesh=plsc.VectorSubcoreMesh(core_axis_name="c", subcore_axis_name="s"),
    out_type=jax.ShapeDtypeStruct((2, N_ACC, ACCW), jnp.float32),
    compiler_params=pltpu.CompilerParams(
        needs_layout_passes=False, use_tc_tiling_on_sc=False),
    scratch_types=(
        [pltpu.VMEM((_B,), jnp.int32),
         pltpu.VMEM((_B,), jnp.int32),
         pltpu.VMEM((_B, _QQW), jnp.float32),
         pltpu.VMEM((_B, HC), jnp.float32),
         pltpu.VMEM((_B, DE), jnp.float32),
         pltpu.VMEM((_B, ACCW), jnp.float32)] * 2
        + [pltpu.VMEM_SHARED((N_ACC, ACCW), jnp.float32)]
        + [pltpu.SemaphoreType.DMA] * 4
    ),
)(_sc_edge_body)


def _edge_pass(qq, ktab, edge_indices, edge_attr):
    src = edge_indices[0]
    trg = edge_indices[1]
    zeros = jnp.zeros((N_ACC, ACCW), jnp.float32)
    return _sc_edge(qq, ktab, src, trg, edge_attr, zeros)


def kernel(x, edge_indices, edge_attr, Wq, bq, Wk, bk, We, Wv, bv,
           Wskip, bskip, Wgate, bgate):
    qq, ktab = _pre_tables(x, Wq, bq, Wk, bk, We)
    acc = _edge_pass(qq, ktab, edge_indices, edge_attr)
    return _post(acc, x, We, Wv, bv, Wskip, bskip, Wgate, bgate)


# speedup vs baseline: 1.5915x; 1.0170x over previous
import functools
import math

import jax
import jax.numpy as jnp
from jax import lax
from jax.experimental import pallas as pl
from jax.experimental.pallas import tpu as pltpu
from jax.experimental.pallas import tpu_sc as plsc

N = 10000
E = 320000
D = 128
H = 4
C = 32
DE = 16
HC = H * C
ACCW = 80
_INV_SQRT_C = 1.0 / math.sqrt(C)

_ROWS_BLK = 2000


def _pre_body(x_ref, wq_ref, bq_ref, wk_ref, bk_ref, we_ref, qq_ref, k_ref):
    x = x_ref[...]
    q = jnp.dot(x, wq_ref[...], preferred_element_type=jnp.float32) + bq_ref[...]
    k = jnp.dot(x, wk_ref[...], preferred_element_type=jnp.float32) + bk_ref[...]
    qq_ref[:, 0:HC] = q
    for h in range(H):
        qh = q[:, C * h:C * h + C]
        we_h = we_ref[:, C * h:C * h + C]
        qq_ref[:, HC + DE * h:HC + DE * (h + 1)] = jax.lax.dot_general(
            qh, we_h, (((1,), (1,)), ((), ())),
            preferred_element_type=jnp.float32)
    k_ref[...] = k


def _pre_tables(x, Wq, bq, Wk, bk, We):
    grid = (N // _ROWS_BLK,)
    return pl.pallas_call(
        _pre_body,
        grid=grid,
        in_specs=[
            pl.BlockSpec((_ROWS_BLK, D), lambda i: (i, 0)),
            pl.BlockSpec((D, HC), lambda i: (0, 0)),
            pl.BlockSpec((1, HC), lambda i: (0, 0)),
            pl.BlockSpec((D, HC), lambda i: (0, 0)),
            pl.BlockSpec((1, HC), lambda i: (0, 0)),
            pl.BlockSpec((DE, HC), lambda i: (0, 0)),
        ],
        out_specs=[
            pl.BlockSpec((_ROWS_BLK, HC + H * DE), lambda i: (i, 0)),
            pl.BlockSpec((_ROWS_BLK, HC), lambda i: (i, 0)),
        ],
        out_shape=[
            jax.ShapeDtypeStruct((N, HC + H * DE), jnp.float32),
            jax.ShapeDtypeStruct((N, HC), jnp.float32),
        ],
    )(x, Wq, bq.reshape(1, HC), Wk, bk.reshape(1, HC), We)


def _post_body(acc_ref, x_ref, we_ref, wv_ref, bv_ref, wskip_ref, bskip_ref,
               g1_ref, g2_ref, bg_ref, out_ref):
    a = acc_ref[0] + acc_ref[1]
    x = x_ref[...]
    v = jnp.dot(x, wv_ref[...], preferred_element_type=jnp.float32) + bv_ref[...]
    skip = jnp.dot(x, wskip_ref[...], preferred_element_type=jnp.float32) + bskip_ref[...]
    parts = []
    for h in range(H):
        s_h = a[:, 16 + DE * h:16 + DE * (h + 1)]
        we_h = we_ref[:, C * h:C * h + C]
        accum_h = jnp.dot(s_h, we_h, preferred_element_type=jnp.float32)
        d_h = a[:, h:h + 1]
        parts.append((v[:, C * h:C * h + C] * d_h + accum_h) / (d_h + 1e-16))
    nf = jnp.concatenate(parts, axis=1)
    glin = (jnp.sum(nf * g1_ref[...], axis=1, keepdims=True)
            + jnp.sum(skip * g2_ref[...], axis=1, keepdims=True)
            + bg_ref[...])
    g = jax.nn.sigmoid(glin)
    out_ref[...] = g * skip + (1.0 - g) * nf


def _post(acc, x, We, Wv, bv, Wskip, bskip, Wgate, bgate):
    g1 = (Wgate[0:HC, 0] - Wgate[2 * HC:3 * HC, 0]).reshape(1, HC)
    g2 = (Wgate[HC:2 * HC, 0] + Wgate[2 * HC:3 * HC, 0]).reshape(1, HC)
    grid = (N // _ROWS_BLK,)
    return pl.pallas_call(
        _post_body,
        grid=grid,
        in_specs=[
            pl.BlockSpec((2, _ROWS_BLK, ACCW), lambda i: (0, i, 0)),
            pl.BlockSpec((_ROWS_BLK, D), lambda i: (i, 0)),
            pl.BlockSpec((DE, HC), lambda i: (0, 0)),
            pl.BlockSpec((D, HC), lambda i: (0, 0)),
            pl.BlockSpec((1, HC), lambda i: (0, 0)),
            pl.BlockSpec((D, HC), lambda i: (0, 0)),
            pl.BlockSpec((1, HC), lambda i: (0, 0)),
            pl.BlockSpec((1, HC), lambda i: (0, 0)),
            pl.BlockSpec((1, HC), lambda i: (0, 0)),
            pl.BlockSpec((1, 1), lambda i: (0, 0)),
        ],
        out_specs=pl.BlockSpec((_ROWS_BLK, HC), lambda i: (i, 0)),
        out_shape=jax.ShapeDtypeStruct((N, HC), jnp.float32),
    )(acc, x, We, Wv, bv.reshape(1, HC), Wskip, bskip.reshape(1, HC),
      g1, g2, bgate.reshape(1, 1))


_B = 80
_NW = 32
_EPW = E // _NW
_NCH = _EPW // _B
N_ACC = 10240
_RPT = N_ACC // 16
_QQW = HC + H * DE


def _sc_edge_body(qq_hbm, k_hbm, src_hbm, trg_hbm, ea_hbm, zero_hbm, out_hbm,
                  src_v0, trg_v0, qq_v0, k_v0, ea_v0, pay_v0,
                  src_v1, trg_v1, qq_v1, k_v1, ea_v1, pay_v1, acc_sh,
                  gsem0, gsem1, ssem0, ssem1):
    c = lax.axis_index("c")
    s = lax.axis_index("s")
    row0 = s * _RPT
    pltpu.sync_copy(zero_hbm.at[pl.ds(row0, _RPT)],
                    acc_sh.at[pl.ds(row0, _RPT)])
    plsc.subcore_barrier()
    lane = lax.iota(jnp.int32, 16)
    eo = lane >> 2
    hh = lane & 3
    colqk0 = hh * C
    colwe0 = HC + hh * DE
    paycol0 = 16 + hh * DE

    def zrow(i, carry):
        z = jnp.zeros((16,), jnp.float32)
        for off in (0, 16, 32, 48, 64):
            pay_v0[i, pl.ds(off, 16)] = z
            pay_v1[i, pl.ds(off, 16)] = z
        return carry

    lax.fori_loop(0, _B, zrow, 0)
    base0 = c * (E // 2) + s * _EPW
    sets = ((src_v0, trg_v0, qq_v0, k_v0, ea_v0, pay_v0, gsem0, ssem0),
            (src_v1, trg_v1, qq_v1, k_v1, ea_v1, pay_v1, gsem1, ssem1))

    def issue_gathers(i, st):
        src_v, trg_v, qq_v, k_v, ea_v, _, gsem, _ = st
        base = base0 + i * _B
        pltpu.sync_copy(src_hbm.at[pl.ds(base, _B)], src_v)
        pltpu.sync_copy(trg_hbm.at[pl.ds(base, _B)], trg_v)
        pltpu.async_copy(qq_hbm.at[src_v], qq_v, gsem)
        pltpu.async_copy(k_hbm.at[trg_v], k_v, gsem)
        pltpu.async_copy(ea_hbm.at[pl.ds(base, _B)], ea_v, gsem)

    def drain_gathers(st):
        src_v, trg_v, qq_v, k_v, ea_v, _, gsem, _ = st
        pltpu.make_async_copy(qq_hbm.at[src_v], qq_v, gsem).wait()
        pltpu.make_async_copy(k_hbm.at[trg_v], k_v, gsem).wait()
        pltpu.make_async_copy(ea_hbm.at[pl.ds(0, _B)], ea_v, gsem).wait()

    def issue_scatter(st):
        _, trg_v, _, _, _, pay_v, _, ssem = st
        pltpu.async_copy(pay_v, acc_sh.at[trg_v], ssem, add=True)

    def drain_scatter(st):
        _, trg_v, _, _, _, pay_v, _, ssem = st
        pltpu.make_async_copy(pay_v, acc_sh.at[trg_v], ssem).wait()

    def compute(st):
        _, _, qq_v, k_v, ea_v, pay_v, _, _ = st

        @plsc.parallel_loop(0, _B // 4, unroll=1)
        def group_body(g):
            row = g * 4 + eo
            acc = jnp.zeros((16,), jnp.float32)
            s32 = lane + 0
            for j in range(C):
                cj = colqk0 + s32
                qv = plsc.load_gather(qq_v, [row, cj])
                kv = plsc.load_gather(k_v, [row, cj])
                acc = acc + qv * kv
                s32 = (s32 + 1) & (C - 1)
            s16 = lane & (DE - 1)
            for d in range(DE):
                qwe = plsc.load_gather(qq_v, [row, colwe0 + s16])
                eav = plsc.load_gather(ea_v, [row, s16])
                acc = acc + qwe * eav
                s16 = (s16 + 1) & (DE - 1)
            w = jnp.exp(acc * _INV_SQRT_C)
            plsc.store_scatter(pay_v, [row, hh], w)
            s16 = lane & (DE - 1)
            for d in range(DE):
                eav = plsc.load_gather(ea_v, [row, s16])
                plsc.store_scatter(pay_v, [row, paycol0 + s16], w * eav)
                s16 = (s16 + 1) & (DE - 1)

    def pair_body(t, carry):
        a = 2 * t

        @pl.when(t > 0)
        def _():
            drain_scatter(sets[0])

        issue_gathers(a, sets[0])

        @pl.when(t > 0)
        def _():
            drain_scatter(sets[1])

        issue_gathers(a + 1, sets[1])
        drain_gathers(sets[0])
        compute(sets[0])
        issue_scatter(sets[0])
        drain_gathers(sets[1])
        compute(sets[1])
        issue_scatter(sets[1])
        return carry

    lax.fori_loop(0, (_NCH - 1) // 2, pair_body, 0)
    drain_scatter(sets[0])
    issue_gathers(_NCH - 1, sets[0])
    drain_gathers(sets[0])
    drain_scatter(sets[1])
    compute(sets[0])
    issue_scatter(sets[0])
    drain_scatter(sets[0])
    plsc.subcore_barrier()
    pltpu.sync_copy(acc_sh.at[pl.ds(row0, _RPT)],
                    out_hbm.at[c, pl.ds(row0, _RPT)])


_sc_edge = functools.partial(
    pl.kernel,
    mesh=plsc.VectorSubcoreMesh(core_axis_name="c", subcore_axis_name="s"),
    out_type=jax.ShapeDtypeStruct((2, N_ACC, ACCW), jnp.float32),
    compiler_params=pltpu.CompilerParams(
        needs_layout_passes=False, use_tc_tiling_on_sc=False),
    scratch_types=(
        [pltpu.VMEM((_B,), jnp.int32),
         pltpu.VMEM((_B,), jnp.int32),
         pltpu.VMEM((_B, _QQW), jnp.float32),
         pltpu.VMEM((_B, HC), jnp.float32),
         pltpu.VMEM((_B, DE), jnp.float32),
         pltpu.VMEM((_B, ACCW), jnp.float32)] * 2
        + [pltpu.VMEM_SHARED((N_ACC, ACCW), jnp.float32)]
        + [pltpu.SemaphoreType.DMA] * 4
    ),
)(_sc_edge_body)


def _edge_pass(qq, ktab, edge_indices, edge_attr):
    src = edge_indices[0]
    trg = edge_indices[1]
    zeros = jnp.zeros((N_ACC, ACCW), jnp.float32)
    return _sc_edge(qq, ktab, src, trg, edge_attr, zeros)


def kernel(x, edge_indices, edge_attr, Wq, bq, Wk, bk, We, Wv, bv,
           Wskip, bskip, Wgate, bgate):
    qq, ktab = _pre_tables(x, Wq, bq, Wk, bk, We)
    acc = _edge_pass(qq, ktab, edge_indices, edge_attr)
    return _post(acc, x, We, Wv, bv, Wskip, bskip, Wgate, bgate)
